# baseline (speedup 1.0000x reference)
import functools

import jax
import jax.numpy as jnp
from jax import lax
from jax.experimental import pallas as pl
from jax.experimental.pallas import tpu as pltpu
from jax.experimental.pallas import tpu_sc as plsc


@functools.lru_cache(maxsize=None)
def _make_sc_gather(B, D):
    info = plsc.get_sparse_core_info()
    NC, NS = info.num_cores, info.num_subcores
    NW = NC * NS
    assert B % NW == 0
    bpw = B // NW
    assert bpw % 8 == 0

    mesh = plsc.VectorSubcoreMesh(core_axis_name="c", subcore_axis_name="s")
    half = bpw // 2

    @functools.partial(
        pl.kernel,
        mesh=mesh,
        compiler_params=pltpu.CompilerParams(use_tc_tiling_on_sc=False),
        out_type=jax.ShapeDtypeStruct((B, D), jnp.bfloat16),
        scratch_types=[
            pltpu.VMEM((bpw,), jnp.int32),
            pltpu.VMEM((half, D), jnp.bfloat16),
            pltpu.VMEM((half, D), jnp.bfloat16),
            pltpu.SemaphoreType.DMA,
            pltpu.SemaphoreType.DMA,
        ],
    )
    def gather(ids, tbl, out, idx, buf0, buf1, s0, s1):
        wid = lax.axis_index("s") * NC + lax.axis_index("c")
        base = wid * bpw
        pltpu.sync_copy(ids.at[pl.ds(base, bpw)], idx)
        c0 = pltpu.async_copy(tbl.at[idx.at[pl.ds(0, half)]], buf0, s0)
        c1 = pltpu.async_copy(tbl.at[idx.at[pl.ds(half, half)]], buf1, s1)
        c0.wait()
        pltpu.sync_copy(buf0, out.at[pl.ds(base, half)])
        c1.wait()
        pltpu.sync_copy(buf1, out.at[pl.ds(base + half, half)])

    return gather


def _dense_body(gu, gi, mu, mi, w1u, w1i, b1, w2t, b2, wlg, wlm, blr, out):
    h = jnp.dot(mu[...].astype(jnp.float32), w1u[...], preferred_element_type=jnp.float32)
    h = h + jnp.dot(mi[...].astype(jnp.float32), w1i[...], preferred_element_type=jnp.float32)
    h = jnp.maximum(h + b1[...], 0.0)
    lm = jnp.dot(h, w2t[...], preferred_element_type=jnp.float32) + b2[...]
    g = gu[...].astype(jnp.float32) * gi[...].astype(jnp.float32)
    s = jnp.sum(g * wlg[...], axis=1, keepdims=True)
    s = s + jnp.sum(lm * wlm[...], axis=1, keepdims=True)
    s = s + blr[...]
    out[...] = 1.0 / (1.0 + jnp.exp(-s))


@functools.lru_cache(maxsize=None)
def _make_tc_dense(B, D, blk):
    grid = (B // blk,)
    full = lambda i: (0, 0)
    return pl.pallas_call(
        _dense_body,
        grid=grid,
        in_specs=[
            pl.BlockSpec((blk, D), lambda i: (i, 0)),
            pl.BlockSpec((blk, D), lambda i: (i, 0)),
            pl.BlockSpec((blk, D), lambda i: (i, 0)),
            pl.BlockSpec((blk, D), lambda i: (i, 0)),
            pl.BlockSpec((D, D), full),
            pl.BlockSpec((D, D), full),
            pl.BlockSpec((1, D), full),
            pl.BlockSpec((D, D), full),
            pl.BlockSpec((1, D), full),
            pl.BlockSpec((1, D), full),
            pl.BlockSpec((1, D), full),
            pl.BlockSpec((1, 1), full),
        ],
        out_specs=pl.BlockSpec((blk, 1), lambda i: (i, 0)),
        out_shape=jax.ShapeDtypeStruct((B, 1), jnp.float32),
    )


def kernel(user_ids, item_ids, U_gmf, I_gmf, U_mlp, I_mlp, W1, b1, W2, b2, Wl, bl):
    B = user_ids.shape[0]
    D = U_gmf.shape[1]
    uids = user_ids.astype(jnp.int32)
    iids = item_ids.astype(jnp.int32)

    g = _make_sc_gather(B, D)
    gu = g(uids, U_gmf.astype(jnp.bfloat16))
    gi = g(iids, I_gmf.astype(jnp.bfloat16))
    mu = g(uids, U_mlp.astype(jnp.bfloat16))
    mi = g(iids, I_mlp.astype(jnp.bfloat16))

    w1t = W1.T
    w1u, w1i = w1t[:D], w1t[D:]
    w2t = W2.T
    wlg, wlm = Wl[:, :D], Wl[:, D:]
    out = _make_tc_dense(B, D, 2048)(
        gu, gi, mu, mi,
        w1u, w1i, b1.reshape(1, D), w2t, b2.reshape(1, D),
        wlg, wlm, bl.reshape(1, 1),
    )
    return out


# speedup vs baseline: 1.6190x; 1.6190x over previous
import functools

import jax
import jax.numpy as jnp
from jax import lax
from jax.experimental import pallas as pl
from jax.experimental.pallas import tpu as pltpu
from jax.experimental.pallas import tpu_sc as plsc


@functools.lru_cache(maxsize=None)
def _make_sc_gather(B, D2):
    info = plsc.get_sparse_core_info()
    NC, NS = info.num_cores, info.num_subcores
    NW = NC * NS
    assert B % NW == 0
    bpw = B // NW
    assert bpw % 8 == 0

    mesh = plsc.VectorSubcoreMesh(core_axis_name="c", subcore_axis_name="s")
    half = bpw // 2

    @functools.partial(
        pl.kernel,
        mesh=mesh,
        compiler_params=pltpu.CompilerParams(use_tc_tiling_on_sc=False),
        out_type=jax.ShapeDtypeStruct((B, D2), jnp.float32),
        scratch_types=[
            pltpu.VMEM((bpw,), jnp.int32),
            pltpu.VMEM((half, D2), jnp.float32),
            pltpu.VMEM((half, D2), jnp.float32),
            pltpu.SemaphoreType.DMA,
            pltpu.SemaphoreType.DMA,
        ],
    )
    def gather(ids, tbl, out, idx, buf0, buf1, s0, s1):
        wid = lax.axis_index("s") * NC + lax.axis_index("c")
        base = wid * bpw
        pltpu.sync_copy(ids.at[pl.ds(base, bpw)], idx)
        c0 = pltpu.async_copy(tbl.at[idx.at[pl.ds(0, half)]], buf0, s0)
        c1 = pltpu.async_copy(tbl.at[idx.at[pl.ds(half, half)]], buf1, s1)
        c0.wait()
        pltpu.sync_copy(buf0, out.at[pl.ds(base, half)])
        c1.wait()
        pltpu.sync_copy(buf1, out.at[pl.ds(base + half, half)])

    return gather


def _dense_body(tu, ti, w1u, w1i, b1, w2t, b2, wlg, wlm, blr, out):
    D = w2t.shape[0]
    gu = tu[:, :D]
    mu = tu[:, D:]
    gi = ti[:, :D]
    mi = ti[:, D:]
    h = jnp.dot(mu, w1u[...], preferred_element_type=jnp.float32)
    h = h + jnp.dot(mi, w1i[...], preferred_element_type=jnp.float32)
    h = jnp.maximum(h + b1[...], 0.0)
    lm = jnp.dot(h, w2t[...], preferred_element_type=jnp.float32) + b2[...]
    g = gu * gi
    s = jnp.sum(g * wlg[...], axis=1, keepdims=True)
    s = s + jnp.sum(lm * wlm[...], axis=1, keepdims=True)
    s = s + blr[...]
    out[...] = 1.0 / (1.0 + jnp.exp(-s))


@functools.lru_cache(maxsize=None)
def _make_tc_dense(B, D, blk):
    grid = (B // blk,)
    full = lambda i: (0, 0)
    return pl.pallas_call(
        _dense_body,
        grid=grid,
        in_specs=[
            pl.BlockSpec((blk, 2 * D), lambda i: (i, 0)),
            pl.BlockSpec((blk, 2 * D), lambda i: (i, 0)),
            pl.BlockSpec((D, D), full),
            pl.BlockSpec((D, D), full),
            pl.BlockSpec((1, D), full),
            pl.BlockSpec((D, D), full),
            pl.BlockSpec((1, D), full),
            pl.BlockSpec((1, D), full),
            pl.BlockSpec((1, D), full),
            pl.BlockSpec((1, 1), full),
        ],
        out_specs=pl.BlockSpec((blk, 1), lambda i: (i, 0)),
        out_shape=jax.ShapeDtypeStruct((B, 1), jnp.float32),
    )


def kernel(user_ids, item_ids, U_gmf, I_gmf, U_mlp, I_mlp, W1, b1, W2, b2, Wl, bl):
    B = user_ids.shape[0]
    D = U_gmf.shape[1]
    uids = user_ids.astype(jnp.int32)
    iids = item_ids.astype(jnp.int32)

    t_u = jnp.concatenate([U_gmf, U_mlp], axis=1)
    t_i = jnp.concatenate([I_gmf, I_mlp], axis=1)

    g = _make_sc_gather(B, 2 * D)
    tu = g(uids, t_u)
    ti = g(iids, t_i)

    w1t = W1.T
    w1u, w1i = w1t[:D], w1t[D:]
    w2t = W2.T
    wlg, wlm = Wl[:, :D], Wl[:, D:]
    out = _make_tc_dense(B, D, 2048)(
        tu, ti,
        w1u, w1i, b1.reshape(1, D), w2t, b2.reshape(1, D),
        wlg, wlm, bl.reshape(1, 1),
    )
    return out
